# Initial kernel scaffold; baseline (speedup 1.0000x reference)
import jax
import jax.numpy as jnp
from jax.experimental import pallas as pl


def kernel(X, beta):
    raise NotImplementedError("write your pallas kernel here")


# speedup vs baseline: 2.9788x; 2.9788x over previous
import jax
import jax.numpy as jnp
from jax import lax
from jax.experimental import pallas as pl
from jax.experimental.pallas import tpu as pltpu
from jax.experimental.pallas import tpu_sc as plsc
import numpy as np

N = 1_000_000
NP = 1_048_576
K = 2048
EPS = 1e-06
NW = 32
PT = NP // NW
VPT = PT // 16
ROWL = 2064
ORL = 2080
MINK = np.int32(-2147483648)


def _gumbel_flat():
    u = jax.random.uniform(jax.random.fold_in(jax.random.key(0), 1), (N,),
                           dtype=jnp.float32)
    return -jnp.log(-jnp.log(u + EPS) + EPS)


def _stage_keys(xt3, g2d, beta2):
    def body(x_ref, g_ref, b_ref, o_ref):
        x = x_ref[...]
        s = x[0]
        for i in range(1, 16):
            s = s + x[i]
        noisy = (s * jnp.float32(0.0625) + g_ref[...]) / b_ref[0, 0]
        b = lax.bitcast_convert_type(noisy, jnp.int32)
        kk = jnp.where(b < 0, jnp.bitwise_xor(jnp.bitwise_not(b), MINK), b)
        o_ref[...] = kk

    return pl.pallas_call(
        body,
        grid=(5,),
        in_specs=[
            pl.BlockSpec((16, 8, 25000), lambda i: (0, i, 0)),
            pl.BlockSpec((8, 25000), lambda i: (i, 0)),
            pl.BlockSpec(memory_space=pltpu.SMEM),
        ],
        out_specs=pl.BlockSpec((8, 25000), lambda i: (i, 0)),
        out_shape=jax.ShapeDtypeStruct((40, 25000), jnp.int32),
    )(xt3, g2d, beta2)


_SC_PARAMS = pltpu.CompilerParams(needs_layout_passes=False)


def _mesh():
    return plsc.VectorSubcoreMesh(core_axis_name="c", subcore_axis_name="s")


def _wid():
    return lax.axis_index("s") * 2 + lax.axis_index("c")


def _load_keys(keys_hbm, kv, wid):
    base = wid * PT
    pltpu.sync_copy(keys_hbm.at[pl.ds(base, PT)], kv)
    return base


def _select_levels(h_hbms, hrows, hg, ss):
    tpart = jnp.zeros((16,), jnp.int32)
    g = jnp.zeros((16,), jnp.int32)
    for l, h_hbm in enumerate(h_hbms):
        pltpu.sync_copy(h_hbm, hrows)

        def comb(c, _):
            def inner(r, acc):
                return acc + hrows[r, pl.ds(c * 16, 16)]
            acc = lax.fori_loop(0, NW, inner, jnp.zeros((16,), jnp.int32))
            hg[pl.ds(c * 16, 16)] = acc
            return 0

        lax.fori_loop(0, 16, comb, 0)

        def sfx(j, carry):
            c = 15 - j
            v = hg[pl.ds(c * 16, 16)]
            within = lax.rev(plsc.cumsum(lax.rev(v, (0,))), (0,))
            ss[pl.ds(c * 16, 16)] = within + carry
            return carry + jnp.sum(v)

        lax.fori_loop(0, 16, sfx, jnp.int32(0))

        need = jnp.int32(K) - g

        def cntm(c, acc):
            m = (ss[pl.ds(c * 16, 16)] >= need).astype(jnp.int32)
            return acc + jnp.sum(m)

        dstar = lax.fori_loop(0, 16, cntm, jnp.int32(0)) - 1
        dsv = jnp.zeros((16,), jnp.int32) + dstar
        g = g + plsc.load_gather(ss, [dsv]) - plsc.load_gather(hg, [dsv])
        dreal = (dsv ^ 0x80) if l == 0 else dsv
        tpart = tpart | (dreal << (24 - 8 * l))
    return tpart, g


def _prefix_mask(k, tpart, level):
    if level == 0:
        return None
    sh = 32 - 8 * level
    return (k >> sh) == (tpart >> sh)


def _make_hist(level):
    n_prev = level

    def body(*refs):
        keys_hbm = refs[0]
        h_prev = refs[1:1 + n_prev]
        h_out = refs[1 + n_prev]
        kv, histloc, hrows, hg, ss, hrow_out = refs[2 + n_prev:]
        wid = _wid()
        _load_keys(keys_hbm, kv, wid)
        tpart, _g = _select_levels(h_prev, hrows, hg, ss)

        def zero(i, _):
            histloc[pl.ds(i * 16, 16)] = jnp.zeros((16,), jnp.int32)
            return 0

        lax.fori_loop(0, 256, zero, 0)
        lane = lax.iota(jnp.int32, 16)
        ones = jnp.ones((16,), jnp.int32)
        sh = 24 - 8 * level

        def scan(i, _):
            k = kv[pl.ds(i * 16, 16)]
            d = (k >> sh) & 0xFF
            if level == 0:
                d = d ^ 0x80
            addr = d * 16 + lane
            m = _prefix_mask(k, tpart, level)
            if m is None:
                m = lane >= 0
            plsc.addupdate_scatter(histloc, [addr], ones, mask=m)
            return 0

        lax.fori_loop(0, VPT, scan, 0)

        def red(c, _):
            idx0 = (c * 16 + lane) * 16

            def gsum(j, acc):
                return acc + plsc.load_gather(histloc, [idx0 + j])

            hrow_out[pl.ds(c * 16, 16)] = lax.fori_loop(
                0, 16, gsum, jnp.zeros((16,), jnp.int32))
            return 0

        lax.fori_loop(0, 16, red, 0)
        pltpu.sync_copy(hrow_out, h_out.at[wid])

    return pl.kernel(
        body,
        out_type=jax.ShapeDtypeStruct((NW, 256), jnp.int32),
        mesh=_mesh(),
        compiler_params=_SC_PARAMS,
        scratch_types=[
            pltpu.VMEM((PT,), jnp.int32),
            pltpu.VMEM((4096,), jnp.int32),
            pltpu.VMEM((NW, 256), jnp.int32),
            pltpu.VMEM((256,), jnp.int32),
            pltpu.VMEM((256,), jnp.int32),
            pltpu.VMEM((256,), jnp.int32),
        ],
    )


def _make_count():
    def body(keys_hbm, h0, h1, h2, h3, c_out, kv, hrows, hg, ss, crow):
        wid = _wid()
        _load_keys(keys_hbm, kv, wid)
        t, _g = _select_levels((h0, h1, h2, h3), hrows, hg, ss)

        def scan(i, carry):
            vg, ve = carry
            k = kv[pl.ds(i * 16, 16)]
            vg = vg + (k > t).astype(jnp.int32)
            ve = ve + (k == t).astype(jnp.int32)
            return vg, ve

        z = jnp.zeros((16,), jnp.int32)
        vg, ve = lax.fori_loop(0, VPT, scan, (z, z))
        lane = lax.iota(jnp.int32, 16)
        crow[...] = jnp.where(lane == 0, jnp.sum(vg),
                              jnp.where(lane == 1, jnp.sum(ve), 0))
        pltpu.sync_copy(crow, c_out.at[wid])

    return pl.kernel(
        body,
        out_type=jax.ShapeDtypeStruct((NW, 16), jnp.int32),
        mesh=_mesh(),
        compiler_params=_SC_PARAMS,
        scratch_types=[
            pltpu.VMEM((PT,), jnp.int32),
            pltpu.VMEM((NW, 256), jnp.int32),
            pltpu.VMEM((256,), jnp.int32),
            pltpu.VMEM((256,), jnp.int32),
            pltpu.VMEM((16,), jnp.int32),
        ],
    )


def _make_compact():
    def body(keys_hbm, h0, h1, h2, h3, cnts, selk, seli, selfx, c2,
             kv, hrows, hg, ss, cl, selk_l, seli_l, selfx_l, c2row):
        wid = _wid()
        base = _load_keys(keys_hbm, kv, wid)
        t, g = _select_levels((h0, h1, h2, h3), hrows, hg, ss)
        r = jnp.int32(K) - g
        pltpu.sync_copy(cnts, cl)
        lane = lax.iota(jnp.int32, 16)
        z16 = jnp.zeros((16,), jnp.int32)

        def pref(i, carry):
            take = (i < wid).astype(jnp.int32)
            row = plsc.load_gather(cl, [z16 + i, lane])
            return carry + take * row

        acc = lax.fori_loop(0, NW, pref, z16)
        epre = acc[1]

        def scan(i, carry):
            csel, ce = carry
            k = kv[pl.ds(i * 16, 16)]
            mg = k > t
            me = k == t
            me_i = me.astype(jnp.int32)
            eqr = epre + ce + plsc.cumsum(me_i) - me_i
            mesel = me & (eqr < r)
            msel = mg | mesel
            ms_i = msel.astype(jnp.int32)
            slot = csel + plsc.cumsum(ms_i) - ms_i
            idxv = base + i * 16 + lane
            fix = jnp.where(mg, jnp.int32(-1), g + eqr)
            plsc.store_scatter(selk_l, [slot], k, mask=msel)
            plsc.store_scatter(seli_l, [slot], idxv, mask=msel)
            plsc.store_scatter(selfx_l, [slot], fix, mask=msel)
            return csel + jnp.sum(ms_i), ce + jnp.sum(me_i)

        csel, _ce = lax.fori_loop(0, VPT, scan, (jnp.int32(0), jnp.int32(0)))
        pltpu.sync_copy(selk_l, selk.at[wid])
        pltpu.sync_copy(seli_l, seli.at[wid])
        pltpu.sync_copy(selfx_l, selfx.at[wid])
        c2row[...] = jnp.where(lane == 0, csel, 0)
        pltpu.sync_copy(c2row, c2.at[wid])

    row = jax.ShapeDtypeStruct((NW, ROWL), jnp.int32)
    return pl.kernel(
        body,
        out_type=(row, row, row, jax.ShapeDtypeStruct((NW, 16), jnp.int32)),
        mesh=_mesh(),
        compiler_params=_SC_PARAMS,
        scratch_types=[
            pltpu.VMEM((PT,), jnp.int32),
            pltpu.VMEM((NW, 256), jnp.int32),
            pltpu.VMEM((256,), jnp.int32),
            pltpu.VMEM((256,), jnp.int32),
            pltpu.VMEM((NW, 16), jnp.int32),
            pltpu.VMEM((ROWL,), jnp.int32),
            pltpu.VMEM((ROWL,), jnp.int32),
            pltpu.VMEM((ROWL,), jnp.int32),
            pltpu.VMEM((16,), jnp.int32),
        ],
    )


def _make_rank():
    def body(selk, seli, selfx, c2, rows_out,
             selk_l, seli_l, selfx_l, cl, orow):
        wid = _wid()
        pltpu.sync_copy(selk, selk_l)
        pltpu.sync_copy(seli.at[wid], seli_l)
        pltpu.sync_copy(selfx.at[wid], selfx_l)
        pltpu.sync_copy(c2, cl)
        lane = lax.iota(jnp.int32, 16)
        z16 = jnp.zeros((16,), jnp.int32)
        widv = z16 + wid
        cnt_own = plsc.load_gather(cl, [widv, z16])[0]

        def initrow(i, _):
            orow[pl.ds(i * 16, 16)] = z16 - 1
            return 0

        lax.fori_loop(0, ORL // 16, initrow, 0)

        def per_avreg(a, _):
            ka = plsc.load_gather(selk_l, [widv, a * 16 + lane])
            qa = wid * ROWL + a * 16 + lane
            cnt = z16
            for tb in range(NW):
                qb0 = tb * ROWL
                tbv = z16 + tb
                cnt_tb = plsc.load_gather(cl, [tbv, z16])[0]

                def per_b(sb, c):
                    kb = plsc.load_gather(selk_l, [tbv, z16 + sb])
                    gt = (kb > ka).astype(jnp.int32)
                    eq = (kb == ka) & ((qb0 + sb) < qa)
                    return c + gt + eq.astype(jnp.int32)

                cnt = lax.fori_loop(0, cnt_tb, per_b, cnt)
            fx = selfx_l[pl.ds(a * 16, 16)]
            pos = jnp.where(fx >= 0, fx, cnt)
            lv = (a * 16 + lane) < cnt_own
            pos = jnp.where(lv, pos, jnp.int32(K) + lane)
            vals = seli_l[pl.ds(a * 16, 16)]
            plsc.store_scatter(orow, [pos], vals)
            return 0

        na = (cnt_own + 15) // 16
        lax.fori_loop(0, na, per_avreg, 0)
        pltpu.sync_copy(orow, rows_out.at[wid])

    return pl.kernel(
        body,
        out_type=jax.ShapeDtypeStruct((NW, ORL), jnp.int32),
        mesh=_mesh(),
        compiler_params=_SC_PARAMS,
        scratch_types=[
            pltpu.VMEM((NW, ROWL), jnp.int32),
            pltpu.VMEM((ROWL,), jnp.int32),
            pltpu.VMEM((ROWL,), jnp.int32),
            pltpu.VMEM((NW, 16), jnp.int32),
            pltpu.VMEM((ORL,), jnp.int32),
        ],
    )


def kernel(X, beta):
    g2d = _gumbel_flat().reshape(40, 25000)
    xt3 = X.T.reshape(16, 40, 25000)
    keys2d = _stage_keys(xt3, g2d, beta.reshape(1, 1))
    keys = jnp.pad(keys2d.reshape(N), (0, NP - N), constant_values=MINK)
    h0 = _make_hist(0)(keys)
    h1 = _make_hist(1)(keys, h0)
    h2 = _make_hist(2)(keys, h0, h1)
    h3 = _make_hist(3)(keys, h0, h1, h2)
    cnts = _make_count()(keys, h0, h1, h2, h3)
    selk, seli, selfx, c2 = _make_compact()(keys, h0, h1, h2, h3, cnts)
    rows = _make_rank()(selk, seli, selfx, c2)
    return jnp.max(rows, axis=0)[:K]


# speedup vs baseline: 3.2674x; 1.0969x over previous
import jax
import jax.numpy as jnp
from jax import lax
from jax.experimental import pallas as pl
from jax.experimental.pallas import tpu as pltpu
from jax.experimental.pallas import tpu_sc as plsc
import numpy as np

N = 1_000_000
NP = 1_048_576
K = 2048
EPS = 1e-06
NW = 32
PT = NP // NW
VPT = PT // 16
UNR = 8
ROWL = 2064
ORL = 2080
MINK = np.int32(-2147483648)


def _gumbel_flat():
    u = jax.random.uniform(jax.random.fold_in(jax.random.key(0), 1), (N,),
                           dtype=jnp.float32)
    return -jnp.log(-jnp.log(u + EPS) + EPS)


def _stage_keys(xt3, g2d, beta2):
    def body(x_ref, g_ref, b_ref, o_ref):
        x = x_ref[...]
        s = x[0]
        for i in range(1, 16):
            s = s + x[i]
        noisy = (s * jnp.float32(0.0625) + g_ref[...]) / b_ref[0, 0]
        b = lax.bitcast_convert_type(noisy, jnp.int32)
        kk = jnp.where(b < 0, jnp.bitwise_xor(jnp.bitwise_not(b), MINK), b)
        o_ref[...] = kk

    return pl.pallas_call(
        body,
        grid=(5,),
        in_specs=[
            pl.BlockSpec((16, 8, 25000), lambda i: (0, i, 0)),
            pl.BlockSpec((8, 25000), lambda i: (i, 0)),
            pl.BlockSpec(memory_space=pltpu.SMEM),
        ],
        out_specs=pl.BlockSpec((8, 25000), lambda i: (i, 0)),
        out_shape=jax.ShapeDtypeStruct((40, 25000), jnp.int32),
    )(xt3, g2d, beta2)


_SC_PARAMS = pltpu.CompilerParams(needs_layout_passes=False)


def _mesh():
    return plsc.VectorSubcoreMesh(core_axis_name="c", subcore_axis_name="s")


def _wid():
    return lax.axis_index("s") * 2 + lax.axis_index("c")


def _z16():
    return jnp.zeros((16,), jnp.int32)


def _meta_vec(tpart, g):
    lane = lax.iota(jnp.int32, 16)
    return jnp.where(lane == 0, tpart, jnp.where(lane == 1, g, 0))


def _meta_read(meta_hbm, mloc, wid):
    del wid
    pltpu.sync_copy(meta_hbm.at[0], mloc)
    v = mloc[pl.ds(0, 16)]
    return _z16() + v[0], _z16() + v[1]


def _combine_select(h_hbm, tpart, g, level, hrows, hg, ss):
    pltpu.sync_copy(h_hbm, hrows)
    for c in range(16):
        acc = hrows[0, pl.ds(c * 16, 16)]
        for r in range(1, NW):
            acc = acc + hrows[r, pl.ds(c * 16, 16)]
        hg[pl.ds(c * 16, 16)] = acc

    def sfx(j, carry):
        c = 15 - j
        v = hg[pl.ds(c * 16, 16)]
        within = lax.rev(plsc.cumsum(lax.rev(v, (0,))), (0,))
        ss[pl.ds(c * 16, 16)] = within + carry
        return carry + jnp.sum(v)

    lax.fori_loop(0, 16, sfx, jnp.int32(0))
    need = jnp.int32(K) - g

    def cntm(c, acc):
        m = (ss[pl.ds(c * 16, 16)] >= need).astype(jnp.int32)
        return acc + jnp.sum(m)

    dstar = lax.fori_loop(0, 16, cntm, jnp.int32(0)) - 1
    dsv = _z16() + dstar
    g = g + plsc.load_gather(ss, [dsv]) - plsc.load_gather(hg, [dsv])
    dreal = (dsv ^ 0x80) if level == 0 else dsv
    tpart = tpart | (dreal << (24 - 8 * level))
    return tpart, g


def _make_hist(level):
    def body(*refs):
        nin = 1 + (1 if level > 0 else 0) + (1 if level > 1 else 0)
        keys_hbm = refs[0]
        h_prev = refs[1] if level > 0 else None
        meta_prev = refs[2] if level > 1 else None
        h_out = refs[nin]
        meta_out = refs[nin + 1] if level > 0 else None
        kv, histloc, hrows, hg, ss, hrow_out, mloc, sem = refs[-8:]
        wid = _wid()
        cp = pltpu.async_copy(keys_hbm.at[pl.ds(wid * PT, PT)], kv, sem)
        if level == 0:
            tpart = _z16()
        else:
            if level > 1:
                tp0, g0 = _meta_read(meta_prev, mloc, wid)
            else:
                tp0, g0 = _z16(), _z16()
            tpart, g = _combine_select(h_prev, tp0, g0, level - 1,
                                       hrows, hg, ss)
            mloc[pl.ds(0, 16)] = _meta_vec(tpart, g)

            @pl.when(wid == 0)
            def _():
                pltpu.sync_copy(mloc, meta_out.at[wid])

        lane = lax.iota(jnp.int32, 16)
        ones = jnp.ones((16,), jnp.int32)

        def zero(i, _):
            histloc[pl.ds(i * 16, 16)] = _z16()
            return 0

        lax.fori_loop(0, 256, zero, 0)
        cp.wait()
        sh = 24 - 8 * level

        def scan(i, _):
            for u in range(UNR):
                k = kv[pl.ds((i * UNR + u) * 16, 16)]
                d = (k >> sh) & 0xFF
                if level == 0:
                    d = d ^ 0x80
                    m = lane >= 0
                else:
                    m = (k >> (32 - 8 * level)) == (tpart >> (32 - 8 * level))
                plsc.addupdate_scatter(histloc, [d * 16 + lane], ones, mask=m)
            return 0

        lax.fori_loop(0, VPT // UNR, scan, 0)

        def red(c, _):
            idx0 = (c * 16 + lane) * 16

            def gsum(j, acc):
                return acc + plsc.load_gather(histloc, [idx0 + j])

            hrow_out[pl.ds(c * 16, 16)] = lax.fori_loop(0, 16, gsum, _z16())
            return 0

        lax.fori_loop(0, 16, red, 0)
        pltpu.sync_copy(hrow_out, h_out.at[wid])

    meta_t = jax.ShapeDtypeStruct((NW, 128), jnp.int32)
    hist_t = jax.ShapeDtypeStruct((NW, 256), jnp.int32)
    return pl.kernel(
        body,
        out_type=hist_t if level == 0 else (hist_t, meta_t),
        mesh=_mesh(),
        compiler_params=_SC_PARAMS,
        scratch_types=[
            pltpu.VMEM((PT,), jnp.int32),
            pltpu.VMEM((4096,), jnp.int32),
            pltpu.VMEM((NW, 256), jnp.int32),
            pltpu.VMEM((256,), jnp.int32),
            pltpu.VMEM((256,), jnp.int32),
            pltpu.VMEM((256,), jnp.int32),
            pltpu.VMEM((128,), jnp.int32),
            pltpu.SemaphoreType.DMA,
        ],
    )


def _make_count():
    def body(keys_hbm, h3, meta2, c_out, meta_out,
             kv, hrows, hg, ss, crow, mloc, sem):
        wid = _wid()
        cp = pltpu.async_copy(keys_hbm.at[pl.ds(wid * PT, PT)], kv, sem)
        tp0, g0 = _meta_read(meta2, mloc, wid)
        t, g = _combine_select(h3, tp0, g0, 3, hrows, hg, ss)
        mloc[pl.ds(0, 16)] = _meta_vec(t, g)

        @pl.when(wid == 0)
        def _():
            pltpu.sync_copy(mloc, meta_out.at[wid])

        cp.wait()

        def scan(i, carry):
            vg, ve = carry
            for u in range(UNR):
                k = kv[pl.ds((i * UNR + u) * 16, 16)]
                vg = vg + (k > t).astype(jnp.int32)
                ve = ve + (k == t).astype(jnp.int32)
            return vg, ve

        vg, ve = lax.fori_loop(0, VPT // UNR, scan, (_z16(), _z16()))
        lane = lax.iota(jnp.int32, 16)
        crow[pl.ds(0, 16)] = jnp.where(lane == 0, jnp.sum(vg),
                               jnp.where(lane == 1, jnp.sum(ve), 0))
        pltpu.sync_copy(crow, c_out.at[wid])

    return pl.kernel(
        body,
        out_type=(jax.ShapeDtypeStruct((NW, 128), jnp.int32),
                  jax.ShapeDtypeStruct((NW, 128), jnp.int32)),
        mesh=_mesh(),
        compiler_params=_SC_PARAMS,
        scratch_types=[
            pltpu.VMEM((PT,), jnp.int32),
            pltpu.VMEM((NW, 256), jnp.int32),
            pltpu.VMEM((256,), jnp.int32),
            pltpu.VMEM((256,), jnp.int32),
            pltpu.VMEM((128,), jnp.int32),
            pltpu.VMEM((128,), jnp.int32),
            pltpu.SemaphoreType.DMA,
        ],
    )


def _make_compact():
    def body(keys_hbm, meta3, cnts, selk, seli, selfx, c2,
             kv, cl, selk_l, seli_l, selfx_l, mloc, c2row, sem):
        wid = _wid()
        cp = pltpu.async_copy(keys_hbm.at[pl.ds(wid * PT, PT)], kv, sem)
        t, g = _meta_read(meta3, mloc, wid)
        r = jnp.int32(K) - g
        pltpu.sync_copy(cnts, cl)
        lane = lax.iota(jnp.int32, 16)
        z16 = _z16()

        def pref(i, carry):
            take = (i < wid).astype(jnp.int32)
            row = plsc.load_gather(cl, [z16 + i, lane])
            return carry + take * row

        acc = lax.fori_loop(0, NW, pref, z16)
        epre = acc[1]

        def fillk(i, _):
            selk_l[pl.ds(i * 16, 16)] = z16 + MINK
            return 0

        lax.fori_loop(0, ROWL // 16, fillk, 0)
        cp.wait()
        base = wid * PT

        def scan(ch, carry):
            ks = [kv[pl.ds((ch * UNR + u) * 16, 16)] for u in range(UNR)]
            m_or = ks[0] >= t
            for u in range(1, UNR):
                m_or = m_or | (ks[u] >= t)
            hit = jnp.sum(m_or.astype(jnp.int32))

            def slow(c):
                csel, ce = c
                for u in range(UNR):
                    k = ks[u]
                    mg = k > t
                    me = k == t
                    me_i = me.astype(jnp.int32)
                    eqr = epre + ce + plsc.cumsum(me_i) - me_i
                    mesel = me & (eqr < r)
                    msel = mg | mesel
                    ms_i = msel.astype(jnp.int32)
                    slot = csel + plsc.cumsum(ms_i) - ms_i
                    idxv = base + (ch * UNR + u) * 16 + lane
                    fix = jnp.where(mg, jnp.int32(-1), g + eqr)
                    plsc.store_scatter(selk_l, [slot], k, mask=msel)
                    plsc.store_scatter(seli_l, [slot], idxv, mask=msel)
                    plsc.store_scatter(selfx_l, [slot], fix, mask=msel)
                    csel = csel + jnp.sum(ms_i)
                    ce = ce + jnp.sum(me_i)
                return csel, ce

            return lax.cond(hit > 0, slow, lambda c: c, carry)

        csel, _ce = lax.fori_loop(0, VPT // UNR, scan,
                                  (jnp.int32(0), jnp.int32(0)))
        pltpu.sync_copy(selk_l, selk.at[wid])
        pltpu.sync_copy(seli_l, seli.at[wid])
        pltpu.sync_copy(selfx_l, selfx.at[wid])
        c2row[pl.ds(0, 16)] = jnp.where(lane == 0, csel, 0)
        pltpu.sync_copy(c2row, c2.at[wid])

    row = jax.ShapeDtypeStruct((NW, ROWL), jnp.int32)
    return pl.kernel(
        body,
        out_type=(row, row, row,
                  jax.ShapeDtypeStruct((NW, 128), jnp.int32)),
        mesh=_mesh(),
        compiler_params=_SC_PARAMS,
        scratch_types=[
            pltpu.VMEM((PT,), jnp.int32),
            pltpu.VMEM((NW, 128), jnp.int32),
            pltpu.VMEM((ROWL,), jnp.int32),
            pltpu.VMEM((ROWL,), jnp.int32),
            pltpu.VMEM((ROWL,), jnp.int32),
            pltpu.VMEM((128,), jnp.int32),
            pltpu.VMEM((128,), jnp.int32),
            pltpu.SemaphoreType.DMA,
        ],
    )


def _make_rank():
    def body(selk, seli, selfx, c2, rows_out,
             selk_l, seli_l, selfx_l, cl, orow):
        wid = _wid()
        pltpu.sync_copy(selk, selk_l)
        pltpu.sync_copy(seli.at[wid], seli_l)
        pltpu.sync_copy(selfx.at[wid], selfx_l)
        pltpu.sync_copy(c2, cl)
        lane = lax.iota(jnp.int32, 16)
        z16 = _z16()
        widv = z16 + wid
        cnt_own = plsc.load_gather(cl, [widv, z16])[0]
        cnt_tb = [plsc.load_gather(cl, [z16 + tb, z16])[0]
                  for tb in range(NW)]

        def initrow(i, _):
            orow[pl.ds(i * 16, 16)] = z16 - 1
            return 0

        lax.fori_loop(0, ORL // 16, initrow, 0)

        def per_apair(a2, _):
            ka0 = plsc.load_gather(selk_l, [widv, a2 * 32 + lane])
            ka1 = plsc.load_gather(selk_l, [widv, a2 * 32 + 16 + lane])
            qa0 = wid * ROWL + a2 * 32 + lane
            qa1 = qa0 + 16
            cnt0 = z16
            cnt1 = z16
            for tb in range(NW):
                qb0 = tb * ROWL
                tbv = z16 + tb

                def per_b8(bb, c):
                    c0, c1 = c
                    for u in range(UNR):
                        sb = bb * UNR + u
                        kb = plsc.load_gather(selk_l, [tbv, z16 + sb])
                        qb = qb0 + sb
                        e0 = (kb == ka0) & (qb < qa0)
                        e1 = (kb == ka1) & (qb < qa1)
                        c0 = c0 + (kb > ka0).astype(jnp.int32) \
                            + e0.astype(jnp.int32)
                        c1 = c1 + (kb > ka1).astype(jnp.int32) \
                            + e1.astype(jnp.int32)
                    return c0, c1

                nb = (cnt_tb[tb] + (UNR - 1)) // UNR
                cnt0, cnt1 = lax.fori_loop(0, nb, per_b8, (cnt0, cnt1))
            for half, (cnt, qoff) in enumerate(((cnt0, 0), (cnt1, 16))):
                off = a2 * 32 + qoff
                fx = selfx_l[pl.ds(off, 16)]
                pos = jnp.where(fx >= 0, fx, cnt)
                lv = (off + lane) < cnt_own
                pos = jnp.where(lv, pos, jnp.int32(K) + lane)
                vals = seli_l[pl.ds(off, 16)]
                plsc.store_scatter(orow, [pos], vals)
            return 0

        na2 = (cnt_own + 31) // 32
        lax.fori_loop(0, na2, per_apair, 0)
        pltpu.sync_copy(orow, rows_out.at[wid])

    return pl.kernel(
        body,
        out_type=jax.ShapeDtypeStruct((NW, ORL), jnp.int32),
        mesh=_mesh(),
        compiler_params=_SC_PARAMS,
        scratch_types=[
            pltpu.VMEM((NW, ROWL), jnp.int32),
            pltpu.VMEM((ROWL,), jnp.int32),
            pltpu.VMEM((ROWL,), jnp.int32),
            pltpu.VMEM((NW, 128), jnp.int32),
            pltpu.VMEM((ORL,), jnp.int32),
        ],
    )


def kernel(X, beta):
    g2d = _gumbel_flat().reshape(40, 25000)
    xt3 = X.T.reshape(16, 40, 25000)
    keys2d = _stage_keys(xt3, g2d, beta.reshape(1, 1))
    keys = jnp.pad(keys2d.reshape(N), (0, NP - N), constant_values=MINK)
    h0 = _make_hist(0)(keys)
    h1, meta0 = _make_hist(1)(keys, h0)
    h2, meta1 = _make_hist(2)(keys, h1, meta0)
    h3, meta2 = _make_hist(3)(keys, h2, meta1)
    cnts, meta3 = _make_count()(keys, h3, meta2)
    selk, seli, selfx, c2 = _make_compact()(keys, meta3, cnts)
    rows = _make_rank()(selk, seli, selfx, c2)
    return jnp.max(rows, axis=0)[:K]


# speedup vs baseline: 3.4970x; 1.0703x over previous
import jax
import jax.numpy as jnp
from jax import lax
from jax.experimental import pallas as pl
from jax.experimental.pallas import tpu as pltpu
from jax.experimental.pallas import tpu_sc as plsc
import numpy as np

N = 1_000_000
NP = 1_048_576
K = 2048
EPS = 1e-06
NW = 32
PT = NP // NW
VPT = PT // 16
UNR = 8
ROWL = 2064
ORL = 2080
MINK = np.int32(-2147483648)


def _gumbel_flat():
    u = jax.random.uniform(jax.random.fold_in(jax.random.key(0), 1), (N,),
                           dtype=jnp.float32)
    return -jnp.log(-jnp.log(u + EPS) + EPS)


def _stage_keys(xt3, g2d, beta2):
    def body(x_ref, g_ref, b_ref, o_ref):
        x = x_ref[...]
        s = x[0]
        for i in range(1, 16):
            s = s + x[i]
        noisy = (s * jnp.float32(0.0625) + g_ref[...]) / b_ref[0, 0]
        b = lax.bitcast_convert_type(noisy, jnp.int32)
        kk = jnp.where(b < 0, jnp.bitwise_xor(jnp.bitwise_not(b), MINK), b)
        o_ref[...] = kk

    return pl.pallas_call(
        body,
        grid=(5,),
        in_specs=[
            pl.BlockSpec((16, 8, 25000), lambda i: (0, i, 0)),
            pl.BlockSpec((8, 25000), lambda i: (i, 0)),
            pl.BlockSpec(memory_space=pltpu.SMEM),
        ],
        out_specs=pl.BlockSpec((8, 25000), lambda i: (i, 0)),
        out_shape=jax.ShapeDtypeStruct((40, 25000), jnp.int32),
    )(xt3, g2d, beta2)


_SC_PARAMS = pltpu.CompilerParams(needs_layout_passes=False)


def _mesh():
    return plsc.VectorSubcoreMesh(core_axis_name="c", subcore_axis_name="s")


def _wid():
    return lax.axis_index("s") * 2 + lax.axis_index("c")


def _z16():
    return jnp.zeros((16,), jnp.int32)


def _meta_vec(tpart, g):
    lane = lax.iota(jnp.int32, 16)
    return jnp.where(lane == 0, tpart, jnp.where(lane == 1, g, 0))


def _meta_read(meta_hbm, mloc, wid):
    del wid
    pltpu.sync_copy(meta_hbm.at[0], mloc)
    v = mloc[pl.ds(0, 16)]
    return _z16() + v[0], _z16() + v[1]


def _combine_select(h_hbm, tpart, g, level, hrows, hg, ss):
    pltpu.sync_copy(h_hbm, hrows)
    for c in range(16):
        acc = hrows[0, pl.ds(c * 16, 16)]
        for r in range(1, NW):
            acc = acc + hrows[r, pl.ds(c * 16, 16)]
        hg[pl.ds(c * 16, 16)] = acc

    def sfx(j, carry):
        c = 15 - j
        v = hg[pl.ds(c * 16, 16)]
        within = lax.rev(plsc.cumsum(lax.rev(v, (0,))), (0,))
        ss[pl.ds(c * 16, 16)] = within + carry
        return carry + jnp.sum(v)

    lax.fori_loop(0, 16, sfx, jnp.int32(0))
    need = jnp.int32(K) - g

    def cntm(c, acc):
        m = (ss[pl.ds(c * 16, 16)] >= need).astype(jnp.int32)
        return acc + jnp.sum(m)

    dstar = lax.fori_loop(0, 16, cntm, jnp.int32(0)) - 1
    dsv = _z16() + dstar
    g = g + plsc.load_gather(ss, [dsv]) - plsc.load_gather(hg, [dsv])
    dreal = (dsv ^ 0x80) if level == 0 else dsv
    tpart = tpart | (dreal << (24 - 8 * level))
    return tpart, g


def _make_hist(level):
    def body(*refs):
        nin = 1 + (1 if level > 0 else 0) + (1 if level > 1 else 0)
        keys_hbm = refs[0]
        h_prev = refs[1] if level > 0 else None
        meta_prev = refs[2] if level > 1 else None
        h_out = refs[nin]
        meta_out = refs[nin + 1] if level > 0 else None
        kv, histloc, hrows, hg, ss, hrow_out, mloc, sem = refs[-8:]
        wid = _wid()
        cp = pltpu.async_copy(keys_hbm.at[pl.ds(wid * PT, PT)], kv, sem)
        if level == 0:
            tpart = _z16()
        else:
            if level > 1:
                tp0, g0 = _meta_read(meta_prev, mloc, wid)
            else:
                tp0, g0 = _z16(), _z16()
            tpart, g = _combine_select(h_prev, tp0, g0, level - 1,
                                       hrows, hg, ss)
            mloc[pl.ds(0, 16)] = _meta_vec(tpart, g)

            @pl.when(wid == 0)
            def _():
                pltpu.sync_copy(mloc, meta_out.at[wid])

        lane = lax.iota(jnp.int32, 16)
        ones = jnp.ones((16,), jnp.int32)

        def zero(i, _):
            histloc[pl.ds(i * 16, 16)] = _z16()
            return 0

        lax.fori_loop(0, 256, zero, 0)
        cp.wait()
        sh = 24 - 8 * level

        def scan(i, _):
            for u in range(UNR):
                k = kv[pl.ds((i * UNR + u) * 16, 16)]
                d = (k >> sh) & 0xFF
                if level == 0:
                    d = d ^ 0x80
                    m = lane >= 0
                else:
                    m = (k >> (32 - 8 * level)) == (tpart >> (32 - 8 * level))
                plsc.addupdate_scatter(histloc, [d * 16 + lane], ones, mask=m)
            return 0

        lax.fori_loop(0, VPT // UNR, scan, 0)

        def red(c, _):
            idx0 = (c * 16 + lane) * 16

            def gsum(j, acc):
                return acc + plsc.load_gather(histloc, [idx0 + j])

            hrow_out[pl.ds(c * 16, 16)] = lax.fori_loop(0, 16, gsum, _z16())
            return 0

        lax.fori_loop(0, 16, red, 0)
        pltpu.sync_copy(hrow_out, h_out.at[wid])

    meta_t = jax.ShapeDtypeStruct((NW, 128), jnp.int32)
    hist_t = jax.ShapeDtypeStruct((NW, 256), jnp.int32)
    return pl.kernel(
        body,
        out_type=hist_t if level == 0 else (hist_t, meta_t),
        mesh=_mesh(),
        compiler_params=_SC_PARAMS,
        scratch_types=[
            pltpu.VMEM((PT,), jnp.int32),
            pltpu.VMEM((4096,), jnp.int32),
            pltpu.VMEM((NW, 256), jnp.int32),
            pltpu.VMEM((256,), jnp.int32),
            pltpu.VMEM((256,), jnp.int32),
            pltpu.VMEM((256,), jnp.int32),
            pltpu.VMEM((128,), jnp.int32),
            pltpu.SemaphoreType.DMA,
        ],
    )


def _make_count():
    def body(keys_hbm, h3, meta2, c_out, meta_out,
             kv, hrows, hg, ss, crow, mloc, sem):
        wid = _wid()
        cp = pltpu.async_copy(keys_hbm.at[pl.ds(wid * PT, PT)], kv, sem)
        tp0, g0 = _meta_read(meta2, mloc, wid)
        t, g = _combine_select(h3, tp0, g0, 3, hrows, hg, ss)
        mloc[pl.ds(0, 16)] = _meta_vec(t, g)

        @pl.when(wid == 0)
        def _():
            pltpu.sync_copy(mloc, meta_out.at[wid])

        cp.wait()

        def scan(i, carry):
            vg, ve = carry
            for u in range(UNR):
                k = kv[pl.ds((i * UNR + u) * 16, 16)]
                vg = vg + (k > t).astype(jnp.int32)
                ve = ve + (k == t).astype(jnp.int32)
            return vg, ve

        vg, ve = lax.fori_loop(0, VPT // UNR, scan, (_z16(), _z16()))
        lane = lax.iota(jnp.int32, 16)
        crow[pl.ds(0, 16)] = jnp.where(lane == 0, jnp.sum(vg),
                               jnp.where(lane == 1, jnp.sum(ve), 0))
        pltpu.sync_copy(crow, c_out.at[wid])

    return pl.kernel(
        body,
        out_type=(jax.ShapeDtypeStruct((NW, 128), jnp.int32),
                  jax.ShapeDtypeStruct((NW, 128), jnp.int32)),
        mesh=_mesh(),
        compiler_params=_SC_PARAMS,
        scratch_types=[
            pltpu.VMEM((PT,), jnp.int32),
            pltpu.VMEM((NW, 256), jnp.int32),
            pltpu.VMEM((256,), jnp.int32),
            pltpu.VMEM((256,), jnp.int32),
            pltpu.VMEM((128,), jnp.int32),
            pltpu.VMEM((128,), jnp.int32),
            pltpu.SemaphoreType.DMA,
        ],
    )


def _make_compact():
    def body(keys_hbm, meta3, cnts, selk, seli, selfx, c2,
             kv, cl, selk_l, seli_l, selfx_l, mloc, c2row, sem):
        wid = _wid()
        cp = pltpu.async_copy(keys_hbm.at[pl.ds(wid * PT, PT)], kv, sem)
        t, g = _meta_read(meta3, mloc, wid)
        r = jnp.int32(K) - g
        pltpu.sync_copy(cnts, cl)
        lane = lax.iota(jnp.int32, 16)
        z16 = _z16()

        def pref(i, carry):
            take = (i < wid).astype(jnp.int32)
            row = plsc.load_gather(cl, [z16 + i, lane])
            return carry + take * row

        acc = lax.fori_loop(0, NW, pref, z16)
        epre = acc[1]

        def fillk(i, _):
            selk_l[pl.ds(i * 16, 16)] = z16 + MINK
            return 0

        lax.fori_loop(0, ROWL // 16, fillk, 0)
        cp.wait()
        base = wid * PT

        def scan(ch, carry):
            ks = [kv[pl.ds((ch * UNR + u) * 16, 16)] for u in range(UNR)]
            m_or = ks[0] >= t
            for u in range(1, UNR):
                m_or = m_or | (ks[u] >= t)
            hit = jnp.sum(m_or.astype(jnp.int32))

            def slow(c):
                csel, ce = c
                for u in range(UNR):
                    k = ks[u]
                    mg = k > t
                    me = k == t
                    me_i = me.astype(jnp.int32)
                    eqr = epre + ce + plsc.cumsum(me_i) - me_i
                    mesel = me & (eqr < r)
                    msel = mg | mesel
                    ms_i = msel.astype(jnp.int32)
                    slot = csel + plsc.cumsum(ms_i) - ms_i
                    idxv = base + (ch * UNR + u) * 16 + lane
                    fix = jnp.where(mg, jnp.int32(-1), g + eqr)
                    plsc.store_scatter(selk_l, [slot], k, mask=msel)
                    plsc.store_scatter(seli_l, [slot], idxv, mask=msel)
                    plsc.store_scatter(selfx_l, [slot], fix, mask=msel)
                    csel = csel + jnp.sum(ms_i)
                    ce = ce + jnp.sum(me_i)
                return csel, ce

            return lax.cond(hit > 0, slow, lambda c: c, carry)

        csel, _ce = lax.fori_loop(0, VPT // UNR, scan,
                                  (jnp.int32(0), jnp.int32(0)))
        pltpu.sync_copy(selk_l, selk.at[wid])
        pltpu.sync_copy(seli_l, seli.at[wid])
        pltpu.sync_copy(selfx_l, selfx.at[wid])
        c2row[pl.ds(0, 16)] = jnp.where(lane == 0, csel, 0)
        pltpu.sync_copy(c2row, c2.at[wid])

    row = jax.ShapeDtypeStruct((NW, ROWL), jnp.int32)
    return pl.kernel(
        body,
        out_type=(row, row, row,
                  jax.ShapeDtypeStruct((NW, 128), jnp.int32)),
        mesh=_mesh(),
        compiler_params=_SC_PARAMS,
        scratch_types=[
            pltpu.VMEM((PT,), jnp.int32),
            pltpu.VMEM((NW, 128), jnp.int32),
            pltpu.VMEM((ROWL,), jnp.int32),
            pltpu.VMEM((ROWL,), jnp.int32),
            pltpu.VMEM((ROWL,), jnp.int32),
            pltpu.VMEM((128,), jnp.int32),
            pltpu.VMEM((128,), jnp.int32),
            pltpu.SemaphoreType.DMA,
        ],
    )


def _make_rank():
    def body(selk, seli, selfx, c2, rows_out,
             selk_l, seli_l, selfx_l, cl, orow):
        wid = _wid()
        pltpu.sync_copy(selk, selk_l)
        pltpu.sync_copy(seli.at[wid], seli_l)
        pltpu.sync_copy(selfx.at[wid], selfx_l)
        pltpu.sync_copy(c2, cl)
        lane = lax.iota(jnp.int32, 16)
        z16 = _z16()
        widv = z16 + wid
        cnt_own = plsc.load_gather(cl, [widv, z16])[0]
        cnt_tb = [plsc.load_gather(cl, [z16 + tb, z16])[0]
                  for tb in range(NW)]

        def initrow(i, _):
            orow[pl.ds(i * 16, 16)] = z16 - 1
            return 0

        lax.fori_loop(0, ORL // 16, initrow, 0)

        def per_avreg(a, _):
            ka = plsc.load_gather(selk_l, [widv, a * 16 + lane])
            qa = wid * ROWL + a * 16 + lane
            cnt = z16
            for tb in range(NW):
                qr = tb * ROWL

                def per_b16(bb, c):
                    kbv = selk_l[tb, pl.ds(bb * 16, 16)]
                    qb0 = qr + bb * 16
                    for u in range(16):
                        kbs = kbv[u]
                        e = (kbs == ka) & ((qb0 + u) < qa)
                        c = c + (kbs > ka).astype(jnp.int32) \
                            + e.astype(jnp.int32)
                    return c

                nb = (cnt_tb[tb] + 15) // 16
                cnt = lax.fori_loop(0, nb, per_b16, cnt)
            fx = selfx_l[pl.ds(a * 16, 16)]
            pos = jnp.where(fx >= 0, fx, cnt)
            lv = (a * 16 + lane) < cnt_own
            pos = jnp.where(lv, pos, jnp.int32(K) + lane)
            vals = seli_l[pl.ds(a * 16, 16)]
            plsc.store_scatter(orow, [pos], vals)
            return 0

        na = (cnt_own + 15) // 16
        lax.fori_loop(0, na, per_avreg, 0)
        pltpu.sync_copy(orow, rows_out.at[wid])

    return pl.kernel(
        body,
        out_type=jax.ShapeDtypeStruct((NW, ORL), jnp.int32),
        mesh=_mesh(),
        compiler_params=_SC_PARAMS,
        scratch_types=[
            pltpu.VMEM((NW, ROWL), jnp.int32),
            pltpu.VMEM((ROWL,), jnp.int32),
            pltpu.VMEM((ROWL,), jnp.int32),
            pltpu.VMEM((NW, 128), jnp.int32),
            pltpu.VMEM((ORL,), jnp.int32),
        ],
    )


def kernel(X, beta):
    g2d = _gumbel_flat().reshape(40, 25000)
    xt3 = X.T.reshape(16, 40, 25000)
    keys2d = _stage_keys(xt3, g2d, beta.reshape(1, 1))
    keys = jnp.pad(keys2d.reshape(N), (0, NP - N), constant_values=MINK)
    h0 = _make_hist(0)(keys)
    h1, meta0 = _make_hist(1)(keys, h0)
    h2, meta1 = _make_hist(2)(keys, h1, meta0)
    h3, meta2 = _make_hist(3)(keys, h2, meta1)
    cnts, meta3 = _make_count()(keys, h3, meta2)
    selk, seli, selfx, c2 = _make_compact()(keys, meta3, cnts)
    rows = _make_rank()(selk, seli, selfx, c2)
    return jnp.max(rows, axis=0)[:K]


# speedup vs baseline: 3.4976x; 1.0002x over previous
import jax
import jax.numpy as jnp
from jax import lax
from jax.experimental import pallas as pl
from jax.experimental.pallas import tpu as pltpu
from jax.experimental.pallas import tpu_sc as plsc
import numpy as np

N = 1_000_000
NP = 1_048_576
K = 2048
EPS = 1e-06
NW = 32
PT = NP // NW
VPT = PT // 16
UNR = 8
ROWL = 2064
ORL = 2080
MINK = np.int32(-2147483648)


def _gumbel_flat():
    u = jax.random.uniform(jax.random.fold_in(jax.random.key(0), 1), (N,),
                           dtype=jnp.float32)
    return -jnp.log(-jnp.log(u + EPS) + EPS)


_GUMBEL = None


def _gumbel_2d():
    global _GUMBEL
    if _GUMBEL is None:
        try:
            _GUMBEL = jax.block_until_ready(
                jax.jit(lambda: _gumbel_flat().reshape(40, 25000))())
        except Exception:
            return _gumbel_flat().reshape(40, 25000)
    return _GUMBEL


def _stage_keys(xt3, g2d, beta2):
    def body(x_ref, g_ref, b_ref, o_ref):
        x = x_ref[...]
        s = x[0]
        for i in range(1, 16):
            s = s + x[i]
        noisy = (s * jnp.float32(0.0625) + g_ref[...]) / b_ref[0, 0]
        b = lax.bitcast_convert_type(noisy, jnp.int32)
        kk = jnp.where(b < 0, jnp.bitwise_xor(jnp.bitwise_not(b), MINK), b)
        o_ref[...] = kk

    return pl.pallas_call(
        body,
        grid=(5,),
        in_specs=[
            pl.BlockSpec((16, 8, 25000), lambda i: (0, i, 0)),
            pl.BlockSpec((8, 25000), lambda i: (i, 0)),
            pl.BlockSpec(memory_space=pltpu.SMEM),
        ],
        out_specs=pl.BlockSpec((8, 25000), lambda i: (i, 0)),
        out_shape=jax.ShapeDtypeStruct((40, 25000), jnp.int32),
    )(xt3, g2d, beta2)


_SC_PARAMS = pltpu.CompilerParams(needs_layout_passes=False)


def _mesh():
    return plsc.VectorSubcoreMesh(core_axis_name="c", subcore_axis_name="s")


def _wid():
    return lax.axis_index("s") * 2 + lax.axis_index("c")


def _z16():
    return jnp.zeros((16,), jnp.int32)


def _meta_vec(tpart, g):
    lane = lax.iota(jnp.int32, 16)
    return jnp.where(lane == 0, tpart, jnp.where(lane == 1, g, 0))


def _meta_read(meta_hbm, mloc, wid):
    del wid
    pltpu.sync_copy(meta_hbm.at[0], mloc)
    v = mloc[pl.ds(0, 16)]
    return _z16() + v[0], _z16() + v[1]


def _combine_select(h_hbm, tpart, g, level, hrows, hg, ss):
    pltpu.sync_copy(h_hbm, hrows)
    for c in range(16):
        acc = hrows[0, pl.ds(c * 16, 16)]
        for r in range(1, NW):
            acc = acc + hrows[r, pl.ds(c * 16, 16)]
        hg[pl.ds(c * 16, 16)] = acc

    def sfx(j, carry):
        c = 15 - j
        v = hg[pl.ds(c * 16, 16)]
        within = lax.rev(plsc.cumsum(lax.rev(v, (0,))), (0,))
        ss[pl.ds(c * 16, 16)] = within + carry
        return carry + jnp.sum(v)

    lax.fori_loop(0, 16, sfx, jnp.int32(0))
    need = jnp.int32(K) - g

    def cntm(c, acc):
        m = (ss[pl.ds(c * 16, 16)] >= need).astype(jnp.int32)
        return acc + jnp.sum(m)

    dstar = lax.fori_loop(0, 16, cntm, jnp.int32(0)) - 1
    dsv = _z16() + dstar
    g = g + plsc.load_gather(ss, [dsv]) - plsc.load_gather(hg, [dsv])
    dreal = (dsv ^ 0x80) if level == 0 else dsv
    tpart = tpart | (dreal << (24 - 8 * level))
    return tpart, g


def _make_hist(level):
    def body(*refs):
        nin = 1 + (1 if level > 0 else 0) + (1 if level > 1 else 0)
        keys_hbm = refs[0]
        h_prev = refs[1] if level > 0 else None
        meta_prev = refs[2] if level > 1 else None
        h_out = refs[nin]
        meta_out = refs[nin + 1] if level > 0 else None
        kv, histloc, hrows, hg, ss, hrow_out, mloc, sem = refs[-8:]
        wid = _wid()
        cp = pltpu.async_copy(keys_hbm.at[pl.ds(wid * PT, PT)], kv, sem)
        if level == 0:
            tpart = _z16()
        else:
            if level > 1:
                tp0, g0 = _meta_read(meta_prev, mloc, wid)
            else:
                tp0, g0 = _z16(), _z16()
            tpart, g = _combine_select(h_prev, tp0, g0, level - 1,
                                       hrows, hg, ss)
            mloc[pl.ds(0, 16)] = _meta_vec(tpart, g)

            @pl.when(wid == 0)
            def _():
                pltpu.sync_copy(mloc, meta_out.at[wid])

        lane = lax.iota(jnp.int32, 16)
        ones = jnp.ones((16,), jnp.int32)

        def zero(i, _):
            histloc[pl.ds(i * 16, 16)] = _z16()
            return 0

        lax.fori_loop(0, 256, zero, 0)
        cp.wait()
        sh = 24 - 8 * level

        def scan(i, _):
            for u in range(UNR):
                k = kv[pl.ds((i * UNR + u) * 16, 16)]
                d = (k >> sh) & 0xFF
                if level == 0:
                    d = d ^ 0x80
                    m = lane >= 0
                else:
                    m = (k >> (32 - 8 * level)) == (tpart >> (32 - 8 * level))
                plsc.addupdate_scatter(histloc, [d * 16 + lane], ones, mask=m)
            return 0

        lax.fori_loop(0, VPT // UNR, scan, 0)

        def red(c, _):
            idx0 = (c * 16 + lane) * 16

            def gsum(j, acc):
                return acc + plsc.load_gather(histloc, [idx0 + j])

            hrow_out[pl.ds(c * 16, 16)] = lax.fori_loop(0, 16, gsum, _z16())
            return 0

        lax.fori_loop(0, 16, red, 0)
        pltpu.sync_copy(hrow_out, h_out.at[wid])

    meta_t = jax.ShapeDtypeStruct((NW, 128), jnp.int32)
    hist_t = jax.ShapeDtypeStruct((NW, 256), jnp.int32)
    return pl.kernel(
        body,
        out_type=hist_t if level == 0 else (hist_t, meta_t),
        mesh=_mesh(),
        compiler_params=_SC_PARAMS,
        scratch_types=[
            pltpu.VMEM((PT,), jnp.int32),
            pltpu.VMEM((4096,), jnp.int32),
            pltpu.VMEM((NW, 256), jnp.int32),
            pltpu.VMEM((256,), jnp.int32),
            pltpu.VMEM((256,), jnp.int32),
            pltpu.VMEM((256,), jnp.int32),
            pltpu.VMEM((128,), jnp.int32),
            pltpu.SemaphoreType.DMA,
        ],
    )


def _make_count():
    def body(keys_hbm, h3, meta2, c_out, meta_out,
             kv, hrows, hg, ss, crow, mloc, sem):
        wid = _wid()
        cp = pltpu.async_copy(keys_hbm.at[pl.ds(wid * PT, PT)], kv, sem)
        tp0, g0 = _meta_read(meta2, mloc, wid)
        t, g = _combine_select(h3, tp0, g0, 3, hrows, hg, ss)
        mloc[pl.ds(0, 16)] = _meta_vec(t, g)

        @pl.when(wid == 0)
        def _():
            pltpu.sync_copy(mloc, meta_out.at[wid])

        cp.wait()

        def scan(i, carry):
            vg, ve = carry
            for u in range(UNR):
                k = kv[pl.ds((i * UNR + u) * 16, 16)]
                vg = vg + (k > t).astype(jnp.int32)
                ve = ve + (k == t).astype(jnp.int32)
            return vg, ve

        vg, ve = lax.fori_loop(0, VPT // UNR, scan, (_z16(), _z16()))
        lane = lax.iota(jnp.int32, 16)
        crow[pl.ds(0, 16)] = jnp.where(lane == 0, jnp.sum(vg),
                               jnp.where(lane == 1, jnp.sum(ve), 0))
        pltpu.sync_copy(crow, c_out.at[wid])

    return pl.kernel(
        body,
        out_type=(jax.ShapeDtypeStruct((NW, 128), jnp.int32),
                  jax.ShapeDtypeStruct((NW, 128), jnp.int32)),
        mesh=_mesh(),
        compiler_params=_SC_PARAMS,
        scratch_types=[
            pltpu.VMEM((PT,), jnp.int32),
            pltpu.VMEM((NW, 256), jnp.int32),
            pltpu.VMEM((256,), jnp.int32),
            pltpu.VMEM((256,), jnp.int32),
            pltpu.VMEM((128,), jnp.int32),
            pltpu.VMEM((128,), jnp.int32),
            pltpu.SemaphoreType.DMA,
        ],
    )


def _make_compact():
    def body(keys_hbm, meta3, cnts, selk, seli, selfx, c2,
             kv, cl, selk_l, seli_l, selfx_l, mloc, c2row, sem):
        wid = _wid()
        cp = pltpu.async_copy(keys_hbm.at[pl.ds(wid * PT, PT)], kv, sem)
        t, g = _meta_read(meta3, mloc, wid)
        r = jnp.int32(K) - g
        pltpu.sync_copy(cnts, cl)
        lane = lax.iota(jnp.int32, 16)
        z16 = _z16()

        def pref(i, carry):
            take = (i < wid).astype(jnp.int32)
            row = plsc.load_gather(cl, [z16 + i, lane])
            return carry + take * row

        acc = lax.fori_loop(0, NW, pref, z16)
        epre = acc[1]

        def fillk(i, _):
            selk_l[pl.ds(i * 16, 16)] = z16 + MINK
            return 0

        lax.fori_loop(0, ROWL // 16, fillk, 0)
        cp.wait()
        base = wid * PT

        def scan(ch, carry):
            ks = [kv[pl.ds((ch * UNR + u) * 16, 16)] for u in range(UNR)]
            m_or = ks[0] >= t
            for u in range(1, UNR):
                m_or = m_or | (ks[u] >= t)
            hit = jnp.sum(m_or.astype(jnp.int32))

            def slow(c):
                csel, ce = c
                for u in range(UNR):
                    k = ks[u]
                    mg = k > t
                    me = k == t
                    me_i = me.astype(jnp.int32)
                    eqr = epre + ce + plsc.cumsum(me_i) - me_i
                    mesel = me & (eqr < r)
                    msel = mg | mesel
                    ms_i = msel.astype(jnp.int32)
                    slot = csel + plsc.cumsum(ms_i) - ms_i
                    idxv = base + (ch * UNR + u) * 16 + lane
                    fix = jnp.where(mg, jnp.int32(-1), g + eqr)
                    plsc.store_scatter(selk_l, [slot], k, mask=msel)
                    plsc.store_scatter(seli_l, [slot], idxv, mask=msel)
                    plsc.store_scatter(selfx_l, [slot], fix, mask=msel)
                    csel = csel + jnp.sum(ms_i)
                    ce = ce + jnp.sum(me_i)
                return csel, ce

            return lax.cond(hit > 0, slow, lambda c: c, carry)

        csel, _ce = lax.fori_loop(0, VPT // UNR, scan,
                                  (jnp.int32(0), jnp.int32(0)))
        pltpu.sync_copy(selk_l, selk.at[wid])
        pltpu.sync_copy(seli_l, seli.at[wid])
        pltpu.sync_copy(selfx_l, selfx.at[wid])
        c2row[pl.ds(0, 16)] = jnp.where(lane == 0, csel, 0)
        pltpu.sync_copy(c2row, c2.at[wid])

    row = jax.ShapeDtypeStruct((NW, ROWL), jnp.int32)
    return pl.kernel(
        body,
        out_type=(row, row, row,
                  jax.ShapeDtypeStruct((NW, 128), jnp.int32)),
        mesh=_mesh(),
        compiler_params=_SC_PARAMS,
        scratch_types=[
            pltpu.VMEM((PT,), jnp.int32),
            pltpu.VMEM((NW, 128), jnp.int32),
            pltpu.VMEM((ROWL,), jnp.int32),
            pltpu.VMEM((ROWL,), jnp.int32),
            pltpu.VMEM((ROWL,), jnp.int32),
            pltpu.VMEM((128,), jnp.int32),
            pltpu.VMEM((128,), jnp.int32),
            pltpu.SemaphoreType.DMA,
        ],
    )


def _make_rank():
    def body(selk, seli, selfx, c2, rows_out,
             selk_l, seli_l, selfx_l, cl, orow):
        wid = _wid()
        pltpu.sync_copy(selk, selk_l)
        pltpu.sync_copy(seli.at[wid], seli_l)
        pltpu.sync_copy(selfx.at[wid], selfx_l)
        pltpu.sync_copy(c2, cl)
        lane = lax.iota(jnp.int32, 16)
        z16 = _z16()
        widv = z16 + wid
        cnt_own = plsc.load_gather(cl, [widv, z16])[0]
        cnt_tb = [plsc.load_gather(cl, [z16 + tb, z16])[0]
                  for tb in range(NW)]

        def initrow(i, _):
            orow[pl.ds(i * 16, 16)] = z16 - 1
            return 0

        lax.fori_loop(0, ORL // 16, initrow, 0)

        def per_avreg(a, _):
            ka = plsc.load_gather(selk_l, [widv, a * 16 + lane])
            qa = wid * ROWL + a * 16 + lane
            cnt = z16
            for tb in range(NW):
                qr = tb * ROWL

                def per_b16(bb, c):
                    kbv = selk_l[tb, pl.ds(bb * 16, 16)]
                    qb0 = qr + bb * 16
                    for u in range(16):
                        kbs = kbv[u]
                        e = (kbs == ka) & ((qb0 + u) < qa)
                        c = c + (kbs > ka).astype(jnp.int32) \
                            + e.astype(jnp.int32)
                    return c

                nb = (cnt_tb[tb] + 15) // 16
                cnt = lax.fori_loop(0, nb, per_b16, cnt)
            fx = selfx_l[pl.ds(a * 16, 16)]
            pos = jnp.where(fx >= 0, fx, cnt)
            lv = (a * 16 + lane) < cnt_own
            pos = jnp.where(lv, pos, jnp.int32(K) + lane)
            vals = seli_l[pl.ds(a * 16, 16)]
            plsc.store_scatter(orow, [pos], vals)
            return 0

        na = (cnt_own + 15) // 16
        lax.fori_loop(0, na, per_avreg, 0)
        pltpu.sync_copy(orow, rows_out.at[wid])

    return pl.kernel(
        body,
        out_type=jax.ShapeDtypeStruct((NW, ORL), jnp.int32),
        mesh=_mesh(),
        compiler_params=_SC_PARAMS,
        scratch_types=[
            pltpu.VMEM((NW, ROWL), jnp.int32),
            pltpu.VMEM((ROWL,), jnp.int32),
            pltpu.VMEM((ROWL,), jnp.int32),
            pltpu.VMEM((NW, 128), jnp.int32),
            pltpu.VMEM((ORL,), jnp.int32),
        ],
    )


def kernel(X, beta):
    g2d = _gumbel_2d()
    xt3 = X.T.reshape(16, 40, 25000)
    keys2d = _stage_keys(xt3, g2d, beta.reshape(1, 1))
    keys = jnp.pad(keys2d.reshape(N), (0, NP - N), constant_values=MINK)
    h0 = _make_hist(0)(keys)
    h1, meta0 = _make_hist(1)(keys, h0)
    h2, meta1 = _make_hist(2)(keys, h1, meta0)
    h3, meta2 = _make_hist(3)(keys, h2, meta1)
    cnts, meta3 = _make_count()(keys, h3, meta2)
    selk, seli, selfx, c2 = _make_compact()(keys, meta3, cnts)
    rows = _make_rank()(selk, seli, selfx, c2)
    return jnp.max(rows, axis=0)[:K]


# speedup vs baseline: 3.4983x; 1.0002x over previous
import jax
import jax.numpy as jnp
from jax import lax
from jax.experimental import pallas as pl
from jax.experimental.pallas import tpu as pltpu
from jax.experimental.pallas import tpu_sc as plsc
import numpy as np

N = 1_000_000
NP = 1_048_576
K = 2048
EPS = 1e-06
NW = 32
PT = NP // NW
VPT = PT // 16
UNR = 8
ROWL = 2064
ORL = 2080
MINK = np.int32(-2147483648)


def _gumbel_flat():
    u = jax.random.uniform(jax.random.fold_in(jax.random.key(0), 1), (N,),
                           dtype=jnp.float32)
    return -jnp.log(-jnp.log(u + EPS) + EPS)


def _stage_keys(xt3, g2d, beta2):
    def body(x_ref, g_ref, b_ref, o_ref):
        x = x_ref[...]
        s = x[0]
        for i in range(1, 16):
            s = s + x[i]
        noisy = (s * jnp.float32(0.0625) + g_ref[...]) / b_ref[0, 0]
        b = lax.bitcast_convert_type(noisy, jnp.int32)
        kk = jnp.where(b < 0, jnp.bitwise_xor(jnp.bitwise_not(b), MINK), b)
        o_ref[...] = kk

    return pl.pallas_call(
        body,
        grid=(5,),
        in_specs=[
            pl.BlockSpec((16, 8, 25000), lambda i: (0, i, 0)),
            pl.BlockSpec((8, 25000), lambda i: (i, 0)),
            pl.BlockSpec(memory_space=pltpu.SMEM),
        ],
        out_specs=pl.BlockSpec((8, 25000), lambda i: (i, 0)),
        out_shape=jax.ShapeDtypeStruct((40, 25000), jnp.int32),
    )(xt3, g2d, beta2)


_SC_PARAMS = pltpu.CompilerParams(needs_layout_passes=False)


def _mesh():
    return plsc.VectorSubcoreMesh(core_axis_name="c", subcore_axis_name="s")


def _wid():
    return lax.axis_index("s") * 2 + lax.axis_index("c")


def _z16():
    return jnp.zeros((16,), jnp.int32)


def _meta_vec(tpart, g):
    lane = lax.iota(jnp.int32, 16)
    return jnp.where(lane == 0, tpart, jnp.where(lane == 1, g, 0))


def _meta_read(meta_hbm, mloc, wid):
    del wid
    pltpu.sync_copy(meta_hbm.at[0], mloc)
    v = mloc[pl.ds(0, 16)]
    return _z16() + v[0], _z16() + v[1]


def _combine_select(h_hbm, tpart, g, level, hrows, hg, ss):
    pltpu.sync_copy(h_hbm, hrows)
    for c in range(16):
        acc = hrows[0, pl.ds(c * 16, 16)]
        for r in range(1, NW):
            acc = acc + hrows[r, pl.ds(c * 16, 16)]
        hg[pl.ds(c * 16, 16)] = acc

    def sfx(j, carry):
        c = 15 - j
        v = hg[pl.ds(c * 16, 16)]
        within = lax.rev(plsc.cumsum(lax.rev(v, (0,))), (0,))
        ss[pl.ds(c * 16, 16)] = within + carry
        return carry + jnp.sum(v)

    lax.fori_loop(0, 16, sfx, jnp.int32(0))
    need = jnp.int32(K) - g

    def cntm(c, acc):
        m = (ss[pl.ds(c * 16, 16)] >= need).astype(jnp.int32)
        return acc + jnp.sum(m)

    dstar = lax.fori_loop(0, 16, cntm, jnp.int32(0)) - 1
    dsv = _z16() + dstar
    g = g + plsc.load_gather(ss, [dsv]) - plsc.load_gather(hg, [dsv])
    dreal = (dsv ^ 0x80) if level == 0 else dsv
    tpart = tpart | (dreal << (24 - 8 * level))
    return tpart, g


def _make_hist(level):
    def body(*refs):
        nin = 1 + (1 if level > 0 else 0) + (1 if level > 1 else 0)
        keys_hbm = refs[0]
        h_prev = refs[1] if level > 0 else None
        meta_prev = refs[2] if level > 1 else None
        h_out = refs[nin]
        meta_out = refs[nin + 1] if level > 0 else None
        kv, histloc, hrows, hg, ss, hrow_out, mloc, sem = refs[-8:]
        wid = _wid()
        cp = pltpu.async_copy(keys_hbm.at[pl.ds(wid * PT, PT)], kv, sem)
        if level == 0:
            tpart = _z16()
        else:
            if level > 1:
                tp0, g0 = _meta_read(meta_prev, mloc, wid)
            else:
                tp0, g0 = _z16(), _z16()
            tpart, g = _combine_select(h_prev, tp0, g0, level - 1,
                                       hrows, hg, ss)
            mloc[pl.ds(0, 16)] = _meta_vec(tpart, g)

            @pl.when(wid == 0)
            def _():
                pltpu.sync_copy(mloc, meta_out.at[wid])

        lane = lax.iota(jnp.int32, 16)
        ones = jnp.ones((16,), jnp.int32)

        def zero(i, _):
            histloc[pl.ds(i * 16, 16)] = _z16()
            return 0

        lax.fori_loop(0, 256, zero, 0)
        cp.wait()
        sh = 24 - 8 * level

        def scan(i, _):
            for u in range(UNR):
                k = kv[pl.ds((i * UNR + u) * 16, 16)]
                d = (k >> sh) & 0xFF
                if level == 0:
                    d = d ^ 0x80
                    m = lane >= 0
                else:
                    m = (k >> (32 - 8 * level)) == (tpart >> (32 - 8 * level))
                plsc.addupdate_scatter(histloc, [d * 16 + lane], ones, mask=m)
            return 0

        lax.fori_loop(0, VPT // UNR, scan, 0)

        def red(c, _):
            idx0 = (c * 16 + lane) * 16

            def gsum(j, acc):
                return acc + plsc.load_gather(histloc, [idx0 + j])

            hrow_out[pl.ds(c * 16, 16)] = lax.fori_loop(0, 16, gsum, _z16())
            return 0

        lax.fori_loop(0, 16, red, 0)
        pltpu.sync_copy(hrow_out, h_out.at[wid])

    meta_t = jax.ShapeDtypeStruct((NW, 128), jnp.int32)
    hist_t = jax.ShapeDtypeStruct((NW, 256), jnp.int32)
    return pl.kernel(
        body,
        out_type=hist_t if level == 0 else (hist_t, meta_t),
        mesh=_mesh(),
        compiler_params=_SC_PARAMS,
        scratch_types=[
            pltpu.VMEM((PT,), jnp.int32),
            pltpu.VMEM((4096,), jnp.int32),
            pltpu.VMEM((NW, 256), jnp.int32),
            pltpu.VMEM((256,), jnp.int32),
            pltpu.VMEM((256,), jnp.int32),
            pltpu.VMEM((256,), jnp.int32),
            pltpu.VMEM((128,), jnp.int32),
            pltpu.SemaphoreType.DMA,
        ],
    )


def _make_count():
    def body(keys_hbm, h3, meta2, c_out, meta_out,
             kv, hrows, hg, ss, crow, mloc, sem):
        wid = _wid()
        cp = pltpu.async_copy(keys_hbm.at[pl.ds(wid * PT, PT)], kv, sem)
        tp0, g0 = _meta_read(meta2, mloc, wid)
        t, g = _combine_select(h3, tp0, g0, 3, hrows, hg, ss)
        mloc[pl.ds(0, 16)] = _meta_vec(t, g)

        @pl.when(wid == 0)
        def _():
            pltpu.sync_copy(mloc, meta_out.at[wid])

        cp.wait()

        def scan(i, carry):
            vg, ve = carry
            for u in range(UNR):
                k = kv[pl.ds((i * UNR + u) * 16, 16)]
                vg = vg + (k > t).astype(jnp.int32)
                ve = ve + (k == t).astype(jnp.int32)
            return vg, ve

        vg, ve = lax.fori_loop(0, VPT // UNR, scan, (_z16(), _z16()))
        lane = lax.iota(jnp.int32, 16)
        crow[pl.ds(0, 16)] = jnp.where(lane == 0, jnp.sum(vg),
                               jnp.where(lane == 1, jnp.sum(ve), 0))
        pltpu.sync_copy(crow, c_out.at[wid])

    return pl.kernel(
        body,
        out_type=(jax.ShapeDtypeStruct((NW, 128), jnp.int32),
                  jax.ShapeDtypeStruct((NW, 128), jnp.int32)),
        mesh=_mesh(),
        compiler_params=_SC_PARAMS,
        scratch_types=[
            pltpu.VMEM((PT,), jnp.int32),
            pltpu.VMEM((NW, 256), jnp.int32),
            pltpu.VMEM((256,), jnp.int32),
            pltpu.VMEM((256,), jnp.int32),
            pltpu.VMEM((128,), jnp.int32),
            pltpu.VMEM((128,), jnp.int32),
            pltpu.SemaphoreType.DMA,
        ],
    )


def _make_compact():
    def body(keys_hbm, meta3, cnts, selk, seli, selfx, c2,
             kv, cl, selk_l, seli_l, selfx_l, mloc, c2row, sem):
        wid = _wid()
        cp = pltpu.async_copy(keys_hbm.at[pl.ds(wid * PT, PT)], kv, sem)
        t, g = _meta_read(meta3, mloc, wid)
        r = jnp.int32(K) - g
        pltpu.sync_copy(cnts, cl)
        lane = lax.iota(jnp.int32, 16)
        z16 = _z16()

        def pref(i, carry):
            take = (i < wid).astype(jnp.int32)
            row = plsc.load_gather(cl, [z16 + i, lane])
            return carry + take * row

        acc = lax.fori_loop(0, NW, pref, z16)
        epre = acc[1]

        def fillk(i, _):
            selk_l[pl.ds(i * 16, 16)] = z16 + MINK
            return 0

        lax.fori_loop(0, ROWL // 16, fillk, 0)
        cp.wait()
        base = wid * PT

        def scan(ch, carry):
            ks = [kv[pl.ds((ch * UNR + u) * 16, 16)] for u in range(UNR)]
            m_or = ks[0] >= t
            for u in range(1, UNR):
                m_or = m_or | (ks[u] >= t)
            hit = jnp.sum(m_or.astype(jnp.int32))

            def slow(c):
                csel, ce = c
                for u in range(UNR):
                    k = ks[u]
                    mg = k > t
                    me = k == t
                    me_i = me.astype(jnp.int32)
                    eqr = epre + ce + plsc.cumsum(me_i) - me_i
                    mesel = me & (eqr < r)
                    msel = mg | mesel
                    ms_i = msel.astype(jnp.int32)
                    slot = csel + plsc.cumsum(ms_i) - ms_i
                    idxv = base + (ch * UNR + u) * 16 + lane
                    fix = jnp.where(mg, jnp.int32(-1), g + eqr)
                    plsc.store_scatter(selk_l, [slot], k, mask=msel)
                    plsc.store_scatter(seli_l, [slot], idxv, mask=msel)
                    plsc.store_scatter(selfx_l, [slot], fix, mask=msel)
                    csel = csel + jnp.sum(ms_i)
                    ce = ce + jnp.sum(me_i)
                return csel, ce

            return lax.cond(hit > 0, slow, lambda c: c, carry)

        csel, _ce = lax.fori_loop(0, VPT // UNR, scan,
                                  (jnp.int32(0), jnp.int32(0)))
        pltpu.sync_copy(selk_l, selk.at[wid])
        pltpu.sync_copy(seli_l, seli.at[wid])
        pltpu.sync_copy(selfx_l, selfx.at[wid])
        c2row[pl.ds(0, 16)] = jnp.where(lane == 0, csel, 0)
        pltpu.sync_copy(c2row, c2.at[wid])

    row = jax.ShapeDtypeStruct((NW, ROWL), jnp.int32)
    return pl.kernel(
        body,
        out_type=(row, row, row,
                  jax.ShapeDtypeStruct((NW, 128), jnp.int32)),
        mesh=_mesh(),
        compiler_params=_SC_PARAMS,
        scratch_types=[
            pltpu.VMEM((PT,), jnp.int32),
            pltpu.VMEM((NW, 128), jnp.int32),
            pltpu.VMEM((ROWL,), jnp.int32),
            pltpu.VMEM((ROWL,), jnp.int32),
            pltpu.VMEM((ROWL,), jnp.int32),
            pltpu.VMEM((128,), jnp.int32),
            pltpu.VMEM((128,), jnp.int32),
            pltpu.SemaphoreType.DMA,
        ],
    )


def _make_rank():
    def body(selk, seli, selfx, c2, rows_out,
             selk_l, seli_l, selfx_l, cl, orow):
        wid = _wid()
        pltpu.sync_copy(selk, selk_l)
        pltpu.sync_copy(seli.at[wid], seli_l)
        pltpu.sync_copy(selfx.at[wid], selfx_l)
        pltpu.sync_copy(c2, cl)
        lane = lax.iota(jnp.int32, 16)
        z16 = _z16()
        widv = z16 + wid
        cnt_own = plsc.load_gather(cl, [widv, z16])[0]
        cnt_tb = [plsc.load_gather(cl, [z16 + tb, z16])[0]
                  for tb in range(NW)]

        def initrow(i, _):
            orow[pl.ds(i * 16, 16)] = z16 - 1
            return 0

        lax.fori_loop(0, ORL // 16, initrow, 0)

        def per_avreg(a, _):
            ka = plsc.load_gather(selk_l, [widv, a * 16 + lane])
            qa = wid * ROWL + a * 16 + lane
            cnt = z16
            for tb in range(NW):
                qr = tb * ROWL

                def per_b16(bb, c):
                    kbv = selk_l[tb, pl.ds(bb * 16, 16)]
                    qb0 = qr + bb * 16
                    for u in range(16):
                        kbs = kbv[u]
                        e = (kbs == ka) & ((qb0 + u) < qa)
                        c = c + (kbs > ka).astype(jnp.int32) \
                            + e.astype(jnp.int32)
                    return c

                nb = (cnt_tb[tb] + 15) // 16
                cnt = lax.fori_loop(0, nb, per_b16, cnt)
            fx = selfx_l[pl.ds(a * 16, 16)]
            pos = jnp.where(fx >= 0, fx, cnt)
            lv = (a * 16 + lane) < cnt_own
            pos = jnp.where(lv, pos, jnp.int32(K) + lane)
            vals = seli_l[pl.ds(a * 16, 16)]
            plsc.store_scatter(orow, [pos], vals)
            return 0

        na = (cnt_own + 15) // 16
        lax.fori_loop(0, na, per_avreg, 0)
        pltpu.sync_copy(orow, rows_out.at[wid])

    return pl.kernel(
        body,
        out_type=jax.ShapeDtypeStruct((NW, ORL), jnp.int32),
        mesh=_mesh(),
        compiler_params=_SC_PARAMS,
        scratch_types=[
            pltpu.VMEM((NW, ROWL), jnp.int32),
            pltpu.VMEM((ROWL,), jnp.int32),
            pltpu.VMEM((ROWL,), jnp.int32),
            pltpu.VMEM((NW, 128), jnp.int32),
            pltpu.VMEM((ORL,), jnp.int32),
        ],
    )


def kernel(X, beta):
    g2d = _gumbel_flat().reshape(40, 25000)
    xt3 = X.T.reshape(16, 40, 25000)
    keys2d = _stage_keys(xt3, g2d, beta.reshape(1, 1))
    keys = jnp.pad(keys2d.reshape(N), (0, NP - N), constant_values=MINK)
    h0 = _make_hist(0)(keys)
    h1, meta0 = _make_hist(1)(keys, h0)
    h2, meta1 = _make_hist(2)(keys, h1, meta0)
    h3, meta2 = _make_hist(3)(keys, h2, meta1)
    cnts, meta3 = _make_count()(keys, h3, meta2)
    selk, seli, selfx, c2 = _make_compact()(keys, meta3, cnts)
    rows = _make_rank()(selk, seli, selfx, c2)
    return jnp.max(rows, axis=0)[:K]
